# baseline (speedup 1.0000x reference)
import functools

import jax
import jax.numpy as jnp
from jax import lax
from jax.experimental import pallas as pl
from jax.experimental.pallas import tpu as pltpu
from jax.experimental.pallas import tpu_sc as plsc

LANES = 16
SEQ_CHUNK = 2
NBUF = 4


@functools.cache
def _build_sc_call(seq: int, batch: int, d: int):
    info = plsc.get_sparse_core_info()
    nw = info.num_cores * info.num_subcores
    seq_per_w = seq // nw
    rows_per_w = seq_per_w * batch
    n_chunks = seq_per_w // SEQ_CHUNK
    rows_per_chunk = SEQ_CHUNK * batch
    assert seq % nw == 0 and seq_per_w % SEQ_CHUNK == 0 and d % LANES == 0
    assert n_chunks % NBUF == 0 and rows_per_chunk % 8 == 0

    mesh = plsc.VectorSubcoreMesh(core_axis_name="c", subcore_axis_name="s")

    @functools.partial(
        pl.kernel,
        mesh=mesh,
        out_type=jax.ShapeDtypeStruct((seq, batch, d), jnp.float32),
        scratch_types=[
            pltpu.VMEM((rows_per_w,), jnp.int32),
            pltpu.VMEM((rows_per_w,), jnp.int32),
            pltpu.VMEM((NBUF, SEQ_CHUNK, batch, d), jnp.float32),
            pltpu.VMEM((NBUF, rows_per_chunk, d), jnp.float32),
            pltpu.VMEM((NBUF, rows_per_chunk, d), jnp.float32),
        ]
        + [pltpu.SemaphoreType.DMA] * (2 * NBUF),
    )
    def sc_kernel(x_hbm, ia_hbm, ib_hbm, pe_hbm, out_hbm,
                  ia_v, ib_v, x_v, ra_v, rb_v, *sems):
        sem_in = sems[:NBUF]
        sem_out = sems[NBUF:]
        wid = lax.axis_index("s") * info.num_cores + lax.axis_index("c")
        row_base_w = wid * rows_per_w
        seq_base_w = wid * seq_per_w

        pltpu.sync_copy(ia_hbm.at[pl.ds(row_base_w, rows_per_w)], ia_v)
        pltpu.sync_copy(ib_hbm.at[pl.ds(row_base_w, rows_per_w)], ib_v)

        def issue_in(ci, b):
            roff = ci * rows_per_chunk
            soff = seq_base_w + ci * SEQ_CHUNK
            pltpu.async_copy(
                pe_hbm.at[ia_v.at[pl.ds(roff, rows_per_chunk)]],
                ra_v.at[b], sem_in[b])
            pltpu.async_copy(
                pe_hbm.at[ib_v.at[pl.ds(roff, rows_per_chunk)]],
                rb_v.at[b], sem_in[b])
            pltpu.async_copy(
                x_hbm.at[pl.ds(soff, SEQ_CHUNK)], x_v.at[b], sem_in[b])

        issue_in(0, 0)

        def super_body(i, carry):
            for b in range(NBUF):
                ci = i * NBUF + b
                bn = (b + 1) % NBUF

                @pl.when(ci >= NBUF - 1)
                def _():
                    pltpu.make_async_copy(
                        x_v.at[bn], out_hbm.at[pl.ds(0, SEQ_CHUNK)],
                        sem_out[bn]).wait()

                @pl.when(ci < n_chunks - 1)
                def _():
                    issue_in(ci + 1, bn)

                pltpu.make_async_copy(
                    x_hbm.at[pl.ds(0, SEQ_CHUNK)], ra_v.at[b],
                    sem_in[b]).wait()
                pltpu.make_async_copy(
                    x_hbm.at[pl.ds(0, SEQ_CHUNK)], rb_v.at[b],
                    sem_in[b]).wait()
                pltpu.make_async_copy(
                    x_hbm.at[pl.ds(0, SEQ_CHUNK)], x_v.at[b],
                    sem_in[b]).wait()

                def seq_body(r, rcarry):
                    for bb in range(batch):
                        for j in range(d // LANES):
                            s = j * LANES
                            row = r * batch + bb
                            val = (ra_v[b, row, pl.ds(s, LANES)]
                                   + rb_v[b, row, pl.ds(s, LANES)]) * 0.001
                            plsc.addupdate(
                                x_v.at[b, r, bb, pl.ds(s, LANES)], val)
                    return rcarry

                lax.fori_loop(0, SEQ_CHUNK, seq_body, 0)

                pltpu.async_copy(
                    x_v.at[b],
                    out_hbm.at[pl.ds(seq_base_w + ci * SEQ_CHUNK, SEQ_CHUNK)],
                    sem_out[b])
            return carry

        lax.fori_loop(0, n_chunks // NBUF, super_body, 0)

        for ci in range(n_chunks - (NBUF - 1), n_chunks):
            b = ci % NBUF
            pltpu.make_async_copy(
                x_v.at[b], out_hbm.at[pl.ds(0, SEQ_CHUNK)], sem_out[b]).wait()

    return sc_kernel


def kernel(x, sec_pos_label, in_sec_pos_label, pe):
    seq, batch, d = x.shape
    n_rows = seq * batch
    ia = sec_pos_label.reshape(n_rows).astype(jnp.int32)
    ib = in_sec_pos_label.reshape(n_rows).astype(jnp.int32)
    pe2 = pe.reshape(pe.shape[0], d)
    return _build_sc_call(seq, batch, d)(x, ia, ib, pe2)


# speedup vs baseline: 1.0483x; 1.0483x over previous
import functools

import jax
import jax.numpy as jnp
from jax import lax
from jax.experimental import pallas as pl
from jax.experimental.pallas import tpu as pltpu
from jax.experimental.pallas import tpu_sc as plsc

LANES = 16
CHUNK = 8
NBUF = 4
NSLICE = 4


@functools.cache
def _build_sc_call(n_rows: int, d: int):
    info = plsc.get_sparse_core_info()
    nw = info.num_cores * info.num_subcores
    rows_per_w = n_rows // nw
    n_chunks = rows_per_w // CHUNK
    assert rows_per_w % CHUNK == 0 and d % LANES == 0
    assert n_chunks % NBUF == 0

    mesh = plsc.VectorSubcoreMesh(core_axis_name="c", subcore_axis_name="s")

    @functools.partial(
        pl.kernel,
        mesh=mesh,
        out_type=jax.ShapeDtypeStruct((n_rows, d), jnp.float32),
        scratch_types=[
            pltpu.VMEM((rows_per_w,), jnp.int32),
            pltpu.VMEM((rows_per_w,), jnp.int32),
            pltpu.VMEM((NBUF, CHUNK, d), jnp.float32),
            pltpu.VMEM((NBUF, CHUNK, d), jnp.float32),
            pltpu.VMEM((NBUF, CHUNK, d), jnp.float32),
        ]
        + [pltpu.SemaphoreType.DMA] * (2 * NBUF),
    )
    def sc_kernel(x_hbm, ia_hbm, ib_hbm, pe_hbm, out_hbm,
                  ia_v, ib_v, x_v, ra_v, rb_v, *sems):
        sem_in = sems[:NBUF]
        sem_out = sems[NBUF:]
        wid = lax.axis_index("s") * info.num_cores + lax.axis_index("c")
        base_w = wid * rows_per_w

        pltpu.sync_copy(ia_hbm.at[pl.ds(base_w, rows_per_w)], ia_v)
        pltpu.sync_copy(ib_hbm.at[pl.ds(base_w, rows_per_w)], ib_v)

        def issue_in(ci, b):
            off = ci * CHUNK
            pltpu.async_copy(
                pe_hbm.at[ia_v.at[pl.ds(off, CHUNK)]], ra_v.at[b], sem_in[b])
            pltpu.async_copy(
                pe_hbm.at[ib_v.at[pl.ds(off, CHUNK)]], rb_v.at[b], sem_in[b])
            pltpu.async_copy(
                x_hbm.at[pl.ds(base_w + off, CHUNK)], x_v.at[b], sem_in[b])

        issue_in(0, 0)

        def super_body(i, carry):
            for b in range(NBUF):
                ci = i * NBUF + b
                bn = (b + 1) % NBUF

                @pl.when(ci >= NBUF - 1)
                def _():
                    pltpu.make_async_copy(
                        x_v.at[bn], out_hbm.at[pl.ds(0, CHUNK)],
                        sem_out[bn]).wait()

                @pl.when(ci < n_chunks - 1)
                def _():
                    issue_in(ci + 1, bn)

                pltpu.make_async_copy(
                    x_hbm.at[pl.ds(0, CHUNK)], ra_v.at[b], sem_in[b]).wait()
                pltpu.make_async_copy(
                    x_hbm.at[pl.ds(0, CHUNK)], rb_v.at[b], sem_in[b]).wait()
                pltpu.make_async_copy(
                    x_hbm.at[pl.ds(0, CHUNK)], x_v.at[b], sem_in[b]).wait()

                def row_body(r, rcarry):
                    for j in range(d // LANES):
                        s = j * LANES
                        val = (ra_v[b, r, pl.ds(s, LANES)]
                               + rb_v[b, r, pl.ds(s, LANES)]) * 0.001
                        plsc.addupdate(x_v.at[b, r, pl.ds(s, LANES)], val)
                    return rcarry

                lax.fori_loop(0, CHUNK, row_body, 0)

                pltpu.async_copy(
                    x_v.at[b],
                    out_hbm.at[pl.ds(base_w + ci * CHUNK, CHUNK)],
                    sem_out[b])
            return carry

        lax.fori_loop(0, n_chunks // NBUF, super_body, 0)

        for ci in range(n_chunks - (NBUF - 1), n_chunks):
            b = ci % NBUF
            pltpu.make_async_copy(
                x_v.at[b], out_hbm.at[pl.ds(0, CHUNK)], sem_out[b]).wait()

    return sc_kernel


def kernel(x, sec_pos_label, in_sec_pos_label, pe):
    seq, batch, d = x.shape
    pe2 = pe.reshape(pe.shape[0], d)
    seq_sl = seq // NSLICE
    rows_sl = seq_sl * batch
    sc_call = _build_sc_call(rows_sl, d)
    outs = []
    for s in range(NSLICE):
        xs = lax.slice_in_dim(x, s * seq_sl, (s + 1) * seq_sl, axis=0)
        ias = lax.slice_in_dim(sec_pos_label, s * seq_sl, (s + 1) * seq_sl,
                               axis=0)
        ibs = lax.slice_in_dim(in_sec_pos_label, s * seq_sl, (s + 1) * seq_sl,
                               axis=0)
        o = sc_call(
            xs.reshape(rows_sl, d),
            ias.reshape(rows_sl).astype(jnp.int32),
            ibs.reshape(rows_sl).astype(jnp.int32),
            pe2,
        )
        outs.append(o.reshape(seq_sl, batch, d))
    return jnp.concatenate(outs, axis=0)


# speedup vs baseline: 1.6877x; 1.6100x over previous
import functools

import jax
import jax.numpy as jnp
from jax import lax
from jax.experimental import pallas as pl
from jax.experimental.pallas import tpu as pltpu
from jax.experimental.pallas import tpu_sc as plsc

LANES = 16
CHUNK = 8
NBUF = 4
AHEAD = 2


@functools.cache
def _build_sc_call(n_rows: int, d: int):
    info = plsc.get_sparse_core_info()
    nw = info.num_cores * info.num_subcores
    rows_per_w = n_rows // nw
    n_chunks = rows_per_w // CHUNK
    assert rows_per_w % CHUNK == 0 and d % LANES == 0
    assert n_chunks % NBUF == 0

    mesh = plsc.VectorSubcoreMesh(core_axis_name="c", subcore_axis_name="s")

    @functools.partial(
        pl.kernel,
        mesh=mesh,
        out_type=jax.ShapeDtypeStruct((n_rows, d), jnp.float32),
        scratch_types=[
            pltpu.VMEM((rows_per_w,), jnp.int32),
            pltpu.VMEM((rows_per_w,), jnp.int32),
            pltpu.VMEM((NBUF, CHUNK, d), jnp.float32),
            pltpu.VMEM((NBUF, CHUNK, d), jnp.float32),
            pltpu.VMEM((NBUF, CHUNK, d), jnp.float32),
        ]
        + [pltpu.SemaphoreType.DMA] * (2 * NBUF),
    )
    def sc_kernel(x_hbm, ia_hbm, ib_hbm, pe_hbm, out_hbm,
                  ia_v, ib_v, x_v, ra_v, rb_v, *sems):
        sem_in = sems[:NBUF]
        sem_out = sems[NBUF:]
        wid = lax.axis_index("s") * info.num_cores + lax.axis_index("c")
        base_w = wid * rows_per_w

        pltpu.sync_copy(ia_hbm.at[pl.ds(base_w, rows_per_w)], ia_v)
        pltpu.sync_copy(ib_hbm.at[pl.ds(base_w, rows_per_w)], ib_v)

        def issue_in(ci, b):
            off = ci * CHUNK
            pltpu.async_copy(
                pe_hbm.at[ia_v.at[pl.ds(off, CHUNK)]], ra_v.at[b], sem_in[b])
            pltpu.async_copy(
                pe_hbm.at[ib_v.at[pl.ds(off, CHUNK)]], rb_v.at[b], sem_in[b])
            pltpu.async_copy(
                x_hbm.at[pl.ds(base_w + off, CHUNK)], x_v.at[b], sem_in[b])

        for p in range(AHEAD):
            issue_in(p, p)

        def super_body(i, carry):
            for b in range(NBUF):
                ci = i * NBUF + b
                bn = (b + AHEAD) % NBUF

                @pl.when(ci >= NBUF - AHEAD)
                def _():
                    pltpu.make_async_copy(
                        x_v.at[bn], out_hbm.at[pl.ds(0, CHUNK)],
                        sem_out[bn]).wait()

                @pl.when(ci < n_chunks - AHEAD)
                def _():
                    issue_in(ci + AHEAD, bn)

                pltpu.make_async_copy(
                    x_hbm.at[pl.ds(0, CHUNK)], ra_v.at[b], sem_in[b]).wait()
                pltpu.make_async_copy(
                    x_hbm.at[pl.ds(0, CHUNK)], rb_v.at[b], sem_in[b]).wait()
                pltpu.make_async_copy(
                    x_hbm.at[pl.ds(0, CHUNK)], x_v.at[b], sem_in[b]).wait()

                def row_body(r, rcarry):
                    for j in range(d // LANES):
                        s = j * LANES
                        val = (ra_v[b, r, pl.ds(s, LANES)]
                               + rb_v[b, r, pl.ds(s, LANES)]) * 0.001
                        plsc.addupdate(x_v.at[b, r, pl.ds(s, LANES)], val)
                    return rcarry

                lax.fori_loop(0, CHUNK, row_body, 0)

                pltpu.async_copy(
                    x_v.at[b],
                    out_hbm.at[pl.ds(base_w + ci * CHUNK, CHUNK)],
                    sem_out[b])
            return carry

        lax.fori_loop(0, n_chunks // NBUF, super_body, 0)

        for ci in range(n_chunks - AHEAD, n_chunks):
            b = ci % NBUF
            pltpu.make_async_copy(
                x_v.at[b], out_hbm.at[pl.ds(0, CHUNK)], sem_out[b]).wait()

    return sc_kernel


def kernel(x, sec_pos_label, in_sec_pos_label, pe):
    seq, batch, d = x.shape
    n_rows = seq * batch
    x2 = x.reshape(n_rows, d)
    ia = sec_pos_label.reshape(n_rows).astype(jnp.int32)
    ib = in_sec_pos_label.reshape(n_rows).astype(jnp.int32)
    pe2 = pe.reshape(pe.shape[0], d)
    out2 = _build_sc_call(n_rows, d)(x2, ia, ib, pe2)
    return out2.reshape(seq, batch, d)


# speedup vs baseline: 2.6717x; 1.5830x over previous
import functools

import jax
import jax.numpy as jnp
from jax import lax
from jax.experimental import pallas as pl
from jax.experimental.pallas import tpu as pltpu
from jax.experimental.pallas import tpu_sc as plsc

LANES = 16
CHUNK = 8
NBUF = 4
AHEAD = 2
SEQ_BLK = 256


@functools.cache
def _build_sc_gather(n_rows: int, d: int):
    info = plsc.get_sparse_core_info()
    nw = info.num_cores * info.num_subcores
    rows_per_w = n_rows // nw
    n_chunks = rows_per_w // CHUNK
    n_ctiles = d // 128
    assert rows_per_w % CHUNK == 0 and d % 128 == 0
    assert n_chunks % NBUF == 0

    mesh = plsc.VectorSubcoreMesh(core_axis_name="c", subcore_axis_name="s")

    @functools.partial(
        pl.kernel,
        mesh=mesh,
        out_type=jax.ShapeDtypeStruct(
            (n_rows // CHUNK, n_ctiles, CHUNK, 128), jnp.float32),
        scratch_types=[
            pltpu.VMEM((rows_per_w,), jnp.int32),
            pltpu.VMEM((rows_per_w,), jnp.int32),
            pltpu.VMEM((NBUF, CHUNK, d), jnp.float32),
            pltpu.VMEM((NBUF, CHUNK, d), jnp.float32),
            pltpu.VMEM((NBUF, 1, n_ctiles, CHUNK, 128), jnp.float32),
        ]
        + [pltpu.SemaphoreType.DMA] * (2 * NBUF),
    )
    def sc_kernel(ia_hbm, ib_hbm, pe_hbm, g_hbm,
                  ia_v, ib_v, ra_v, rb_v, o_v, *sems):
        sem_in = sems[:NBUF]
        sem_out = sems[NBUF:]
        wid = lax.axis_index("s") * info.num_cores + lax.axis_index("c")
        base_w = wid * rows_per_w
        rtile_w = wid * n_chunks

        pltpu.sync_copy(ia_hbm.at[pl.ds(base_w, rows_per_w)], ia_v)
        pltpu.sync_copy(ib_hbm.at[pl.ds(base_w, rows_per_w)], ib_v)

        def issue_in(ci, b):
            off = ci * CHUNK
            pltpu.async_copy(
                pe_hbm.at[ia_v.at[pl.ds(off, CHUNK)]], ra_v.at[b], sem_in[b])
            pltpu.async_copy(
                pe_hbm.at[ib_v.at[pl.ds(off, CHUNK)]], rb_v.at[b], sem_in[b])

        for p in range(AHEAD):
            issue_in(p, p)

        def super_body(i, carry):
            for b in range(NBUF):
                ci = i * NBUF + b
                bn = (b + AHEAD) % NBUF

                @pl.when(ci >= NBUF - AHEAD)
                def _():
                    pltpu.make_async_copy(
                        o_v.at[bn], g_hbm.at[pl.ds(0, 1)],
                        sem_out[bn]).wait()

                @pl.when(ci < n_chunks - AHEAD)
                def _():
                    issue_in(ci + AHEAD, bn)

                pltpu.make_async_copy(
                    pe_hbm.at[pl.ds(0, CHUNK)], ra_v.at[b],
                    sem_in[b]).wait()
                pltpu.make_async_copy(
                    pe_hbm.at[pl.ds(0, CHUNK)], rb_v.at[b],
                    sem_in[b]).wait()

                def row_body(r, rcarry):
                    for j in range(d // LANES):
                        c, l = j // 8, (j % 8) * LANES
                        s = j * LANES
                        o_v[b, 0, c, r, pl.ds(l, LANES)] = (
                            ra_v[b, r, pl.ds(s, LANES)]
                            + rb_v[b, r, pl.ds(s, LANES)]) * 0.001
                    return rcarry

                lax.fori_loop(0, CHUNK, row_body, 0)

                pltpu.async_copy(
                    o_v.at[b], g_hbm.at[pl.ds(rtile_w + ci, 1)], sem_out[b])
            return carry

        lax.fori_loop(0, n_chunks // NBUF, super_body, 0)

        for ci in range(n_chunks - AHEAD, n_chunks):
            b = ci % NBUF
            pltpu.make_async_copy(
                o_v.at[b], g_hbm.at[pl.ds(0, 1)], sem_out[b]).wait()

    return sc_kernel


def _tc_add_body(x_ref, g_ref, o_ref):
    gb = g_ref[...]
    gt = jnp.transpose(gb, (0, 2, 1, 3))
    nrows = gb.shape[0] * 8
    s, bt, d = x_ref.shape
    g3 = gt.reshape(nrows, d).reshape(s, bt, d)
    o_ref[...] = x_ref[...] + g3


@functools.cache
def _build_tc_add(seq: int, batch: int, d: int):
    n_rblk = SEQ_BLK * batch // 8
    return pl.pallas_call(
        _tc_add_body,
        grid=(seq // SEQ_BLK,),
        in_specs=[
            pl.BlockSpec((SEQ_BLK, batch, d), lambda i: (i, 0, 0)),
            pl.BlockSpec((n_rblk, d // 128, 8, 128), lambda i: (i, 0, 0, 0)),
        ],
        out_specs=pl.BlockSpec((SEQ_BLK, batch, d), lambda i: (i, 0, 0)),
        out_shape=jax.ShapeDtypeStruct((seq, batch, d), jnp.float32),
    )


def kernel(x, sec_pos_label, in_sec_pos_label, pe):
    seq, batch, d = x.shape
    n_rows = seq * batch
    ia = sec_pos_label.reshape(n_rows).astype(jnp.int32)
    ib = in_sec_pos_label.reshape(n_rows).astype(jnp.int32)
    pe2 = pe.reshape(pe.shape[0], d)
    g = _build_sc_gather(n_rows, d)(ia, ib, pe2)
    return _build_tc_add(seq, batch, d)(x, g)


# speedup vs baseline: 2.6740x; 1.0009x over previous
import functools

import jax
import jax.numpy as jnp
from jax import lax
from jax.experimental import pallas as pl
from jax.experimental.pallas import tpu as pltpu
from jax.experimental.pallas import tpu_sc as plsc

LANES = 16
CHUNK = 8
NBUF = 4
AHEAD = 2
SEQ_BLK = 512


@functools.cache
def _build_sc_gather(n_rows: int, d: int):
    info = plsc.get_sparse_core_info()
    nw = info.num_cores * info.num_subcores
    rows_per_w = n_rows // nw
    n_chunks = rows_per_w // CHUNK
    n_ctiles = d // 128
    assert rows_per_w % CHUNK == 0 and d % 128 == 0
    assert n_chunks % NBUF == 0

    mesh = plsc.VectorSubcoreMesh(core_axis_name="c", subcore_axis_name="s")

    @functools.partial(
        pl.kernel,
        mesh=mesh,
        out_type=jax.ShapeDtypeStruct(
            (n_rows // CHUNK, n_ctiles, CHUNK, 128), jnp.float32),
        scratch_types=[
            pltpu.VMEM((rows_per_w,), jnp.int32),
            pltpu.VMEM((rows_per_w,), jnp.int32),
            pltpu.VMEM((NBUF, CHUNK, d), jnp.float32),
            pltpu.VMEM((NBUF, CHUNK, d), jnp.float32),
            pltpu.VMEM((NBUF, 1, n_ctiles, CHUNK, 128), jnp.float32),
        ]
        + [pltpu.SemaphoreType.DMA] * (2 * NBUF),
    )
    def sc_kernel(ia_hbm, ib_hbm, pe_hbm, g_hbm,
                  ia_v, ib_v, ra_v, rb_v, o_v, *sems):
        sem_in = sems[:NBUF]
        sem_out = sems[NBUF:]
        wid = lax.axis_index("s") * info.num_cores + lax.axis_index("c")
        base_w = wid * rows_per_w
        rtile_w = wid * n_chunks

        pltpu.sync_copy(ia_hbm.at[pl.ds(base_w, rows_per_w)], ia_v)
        pltpu.sync_copy(ib_hbm.at[pl.ds(base_w, rows_per_w)], ib_v)

        def issue_in(ci, b):
            off = ci * CHUNK
            pltpu.async_copy(
                pe_hbm.at[ia_v.at[pl.ds(off, CHUNK)]], ra_v.at[b], sem_in[b])
            pltpu.async_copy(
                pe_hbm.at[ib_v.at[pl.ds(off, CHUNK)]], rb_v.at[b], sem_in[b])

        for p in range(AHEAD):
            issue_in(p, p)

        def super_body(i, carry):
            for b in range(NBUF):
                ci = i * NBUF + b
                bn = (b + AHEAD) % NBUF

                @pl.when(ci >= NBUF - AHEAD)
                def _():
                    pltpu.make_async_copy(
                        o_v.at[bn], g_hbm.at[pl.ds(0, 1)],
                        sem_out[bn]).wait()

                @pl.when(ci < n_chunks - AHEAD)
                def _():
                    issue_in(ci + AHEAD, bn)

                pltpu.make_async_copy(
                    pe_hbm.at[pl.ds(0, CHUNK)], ra_v.at[b],
                    sem_in[b]).wait()
                pltpu.make_async_copy(
                    pe_hbm.at[pl.ds(0, CHUNK)], rb_v.at[b],
                    sem_in[b]).wait()

                def row_body(r, rcarry):
                    for j in range(d // LANES):
                        c, l = j // 8, (j % 8) * LANES
                        s = j * LANES
                        o_v[b, 0, c, r, pl.ds(l, LANES)] = (
                            ra_v[b, r, pl.ds(s, LANES)]
                            + rb_v[b, r, pl.ds(s, LANES)]) * 0.001
                    return rcarry

                lax.fori_loop(0, CHUNK, row_body, 0)

                pltpu.async_copy(
                    o_v.at[b], g_hbm.at[pl.ds(rtile_w + ci, 1)], sem_out[b])
            return carry

        lax.fori_loop(0, n_chunks // NBUF, super_body, 0)

        for ci in range(n_chunks - AHEAD, n_chunks):
            b = ci % NBUF
            pltpu.make_async_copy(
                o_v.at[b], g_hbm.at[pl.ds(0, 1)], sem_out[b]).wait()

    return sc_kernel


def _tc_add_body(x_ref, g_ref, o_ref):
    gb = g_ref[...]
    gt = jnp.transpose(gb, (0, 2, 1, 3))
    nrows = gb.shape[0] * 8
    s, bt, d = x_ref.shape
    g3 = gt.reshape(nrows, d).reshape(s, bt, d)
    o_ref[...] = x_ref[...] + g3


@functools.cache
def _build_tc_add(seq: int, batch: int, d: int):
    n_rblk = SEQ_BLK * batch // 8
    return pl.pallas_call(
        _tc_add_body,
        grid=(seq // SEQ_BLK,),
        in_specs=[
            pl.BlockSpec((SEQ_BLK, batch, d), lambda i: (i, 0, 0)),
            pl.BlockSpec((n_rblk, d // 128, 8, 128), lambda i: (i, 0, 0, 0)),
        ],
        out_specs=pl.BlockSpec((SEQ_BLK, batch, d), lambda i: (i, 0, 0)),
        out_shape=jax.ShapeDtypeStruct((seq, batch, d), jnp.float32),
    )


def kernel(x, sec_pos_label, in_sec_pos_label, pe):
    seq, batch, d = x.shape
    n_rows = seq * batch
    ia = sec_pos_label.reshape(n_rows).astype(jnp.int32)
    ib = in_sec_pos_label.reshape(n_rows).astype(jnp.int32)
    pe2 = pe.reshape(pe.shape[0], d)
    g = _build_sc_gather(n_rows, d)(ia, ib, pe2)
    return _build_tc_add(seq, batch, d)(x, g)
